# Initial kernel scaffold; baseline (speedup 1.0000x reference)
import jax
import jax.numpy as jnp
from jax.experimental import pallas as pl


def kernel(x, emb_table, pos_table):
    raise NotImplementedError("write your pallas kernel here")


# speedup vs baseline: 3.1067x; 3.1067x over previous
import functools

import jax
import jax.numpy as jnp
from jax import lax
from jax.experimental import pallas as pl
from jax.experimental.pallas import tpu as pltpu
from jax.experimental.pallas import tpu_sc as plsc

_NC, _NS, _LANES = 2, 16, 16


@functools.partial(jax.jit, static_argnums=())
def kernel(x, emb_table, pos_table):
    B, L = x.shape
    V, H = emb_table.shape
    x_flat = x.reshape(B * L).astype(jnp.int32)
    pos = pos_table[:L].astype(jnp.float32)
    out = _build(B, L, H)(x_flat, emb_table, pos)
    return out.reshape(B, L, H)


@functools.lru_cache(maxsize=None)
def _build(B, L, H):
    NW = _NC * _NS
    assert B % NW == 0, (B, NW)
    b_per_w = B // NW
    assert H % _LANES == 0
    mesh = plsc.VectorSubcoreMesh(core_axis_name="c", subcore_axis_name="s")

    @functools.partial(
        pl.kernel,
        out_type=jax.ShapeDtypeStruct((B * L, H), jnp.float32),
        mesh=mesh,
        scratch_types=[
            pltpu.VMEM((L, H), jnp.float32),
            pltpu.VMEM((L,), jnp.int32),
            pltpu.VMEM((L, H), jnp.float32),
            pltpu.SemaphoreType.DMA,
        ],
        compiler_params=pltpu.CompilerParams(use_tc_tiling_on_sc=False),
    )
    def k(x_hbm, emb_hbm, pos_hbm, out_hbm, pos_v, idx_v, rows_v, sem):
        wid = lax.axis_index("s") * _NC + lax.axis_index("c")
        pltpu.sync_copy(pos_hbm, pos_v)
        base = wid * b_per_w * L

        def chunk(i, carry):
            rb = base + i * L
            pltpu.sync_copy(x_hbm.at[pl.ds(rb, L)], idx_v)
            cps = []
            for off in range(0, L, 128):
                n = min(128, L - off)
                cps.append(pltpu.async_copy(
                    emb_hbm.at[idx_v.at[pl.ds(off, n)]],
                    rows_v.at[pl.ds(off, n)],
                    sem,
                ))
            for cp in cps:
                cp.wait()

            def add_row(r, c2):
                for j in range(H // _LANES):
                    v = pos_v[r, pl.ds(j * _LANES, _LANES)]
                    plsc.addupdate(rows_v.at[r, pl.ds(j * _LANES, _LANES)], v)
                return c2

            lax.fori_loop(0, L, add_row, 0, unroll=4)
            pltpu.sync_copy(rows_v, out_hbm.at[pl.ds(rb, L)])
            return carry

        lax.fori_loop(0, b_per_w, chunk, 0)

    return k


# speedup vs baseline: 4.2141x; 1.3564x over previous
import functools

import jax
import jax.numpy as jnp
from jax import lax
from jax.experimental import pallas as pl
from jax.experimental.pallas import tpu as pltpu
from jax.experimental.pallas import tpu_sc as plsc

_NC, _NS, _LANES = 2, 16, 16
_NBUF = 4


@jax.jit
def kernel(x, emb_table, pos_table):
    B, L = x.shape
    V, H = emb_table.shape
    x_flat = x.reshape(B * L).astype(jnp.int32)
    pos = pos_table[:L].astype(jnp.float32)
    out = _build(B, L, H)(x_flat, emb_table, pos)
    return out.reshape(B, L, H)


@functools.lru_cache(maxsize=None)
def _build(B, L, H):
    NW = _NC * _NS
    assert B % NW == 0, (B, NW)
    n_chunks = B // NW
    assert n_chunks >= 2 * _NBUF
    assert (n_chunks - 1 - (_NBUF - 1)) % _NBUF == 0
    assert H % _LANES == 0
    mesh = plsc.VectorSubcoreMesh(core_axis_name="c", subcore_axis_name="s")

    scratch = [
        pltpu.VMEM((L, H), jnp.float32),
        pltpu.VMEM((_NBUF, L), jnp.int32),
        pltpu.VMEM((_NBUF, L, H), jnp.float32)
    ] + [pltpu.SemaphoreType.DMA] * (3 * _NBUF)

    @functools.partial(
        pl.kernel,
        out_type=jax.ShapeDtypeStruct((B * L, H), jnp.float32),
        mesh=mesh,
        scratch_types=scratch,
        compiler_params=pltpu.CompilerParams(use_tc_tiling_on_sc=False),
    )
    def k(x_hbm, emb_hbm, pos_hbm, out_hbm, pos_v, idx_v, rows_v, *sems):
        sem_i = sems[0:_NBUF]
        sem_g = sems[_NBUF:2 * _NBUF]
        sem_s = sems[2 * _NBUF:3 * _NBUF]
        wid = lax.axis_index("s") * _NC + lax.axis_index("c")
        pltpu.sync_copy(pos_hbm, pos_v)
        base = wid * n_chunks * L

        def fire_idx(i, s):
            pltpu.async_copy(x_hbm.at[pl.ds(base + i * L, L)], idx_v.at[s],
                             sem_i[s])

        def wait_idx(s):
            pltpu.make_async_copy(x_hbm.at[pl.ds(0, L)], idx_v.at[s],
                                  sem_i[s]).wait()

        def fire_gather(s):
            for off in range(0, L, 128):
                n = min(128, L - off)
                pltpu.async_copy(emb_hbm.at[idx_v.at[s, pl.ds(off, n)]],
                                 rows_v.at[s, pl.ds(off, n)], sem_g[s])

        def wait_gather(s):
            pltpu.make_async_copy(out_hbm.at[pl.ds(0, L)], rows_v.at[s],
                                  sem_g[s]).wait()

        def fire_store(i, s):
            pltpu.async_copy(rows_v.at[s], out_hbm.at[pl.ds(base + i * L, L)],
                             sem_s[s])

        def wait_store(s):
            pltpu.make_async_copy(rows_v.at[s], out_hbm.at[pl.ds(0, L)],
                                  sem_s[s]).wait()

        def add_pos(s):
            def add_row(r, c):
                for j in range(H // _LANES):
                    v = pos_v[r, pl.ds(j * _LANES, _LANES)]
                    plsc.addupdate(rows_v.at[s, r, pl.ds(j * _LANES, _LANES)],
                                   v)
                return c

            lax.fori_loop(0, L, add_row, 0, unroll=4)

        for j in range(_NBUF - 1):
            fire_idx(j, j)
            wait_idx(j)
            fire_gather(j)

        def step(i, s, prefetch, first=False):
            sp = (s + _NBUF - 1) % _NBUF
            if prefetch:
                fire_idx(i + (_NBUF - 1), sp)
            wait_gather(s)
            add_pos(s)
            fire_store(i, s)
            if prefetch:
                if not first:
                    wait_store(sp)
                wait_idx(sp)
                fire_gather(sp)

        step(0, 0, prefetch=True, first=True)

        n_main = n_chunks - 1 - (_NBUF - 1)

        def main_body(t, c):
            for u in range(_NBUF):
                i = 1 + t * _NBUF + u
                step(i, (1 + u) % _NBUF, prefetch=True)
            return c

        lax.fori_loop(0, n_main // _NBUF, main_body, 0)

        for i in range(n_chunks - (_NBUF - 1), n_chunks):
            step(i, i % _NBUF, prefetch=False)

        for s in range(_NBUF):
            wait_store(s)

    return k


# speedup vs baseline: 4.2384x; 1.0058x over previous
import functools

import jax
import jax.numpy as jnp
from jax import lax
from jax.experimental import pallas as pl
from jax.experimental.layout import Format, Layout
from jax.experimental.pallas import tpu as pltpu
from jax.experimental.pallas import tpu_sc as plsc
from jax.sharding import SingleDeviceSharding

_NC, _NS, _LANES = 2, 16, 16
_NBUF = 4


def _kernel_impl(x, emb_table, pos_table):
    B, L = x.shape
    V, H = emb_table.shape
    x_flat = x.reshape(B * L).astype(jnp.int32)
    pos = pos_table[:L].astype(jnp.float32)
    return _build(B, L, H)(x_flat, emb_table, pos)


_kernel_impl.__name__ = "kernel"

_jitted = None


def kernel(x, emb_table, pos_table):
    global _jitted
    if _jitted is None:
        try:
            dev = next(iter(x.devices()))
        except Exception:
            dev = jax.devices()[0]
        fmt = Format(Layout(major_to_minor=(0, 1, 2), tiling=((8,),)),
                     SingleDeviceSharding(dev))
        _jitted = jax.jit(_kernel_impl, out_shardings=fmt)
    return _jitted(x, emb_table, pos_table)


@functools.lru_cache(maxsize=None)
def _build(B, L, H):
    NW = _NC * _NS
    assert B % NW == 0, (B, NW)
    n_chunks = B // NW
    assert n_chunks >= 2 * _NBUF
    assert (n_chunks - 1 - (_NBUF - 1)) % _NBUF == 0
    assert H % _LANES == 0
    mesh = plsc.VectorSubcoreMesh(core_axis_name="c", subcore_axis_name="s")

    scratch = [
        pltpu.VMEM((L, H), jnp.float32),
        pltpu.VMEM((_NBUF, L), jnp.int32),
        pltpu.VMEM((_NBUF, L, H), jnp.float32)
    ] + [pltpu.SemaphoreType.DMA] * (3 * _NBUF)

    @functools.partial(
        pl.kernel,
        out_type=jax.ShapeDtypeStruct((B, L, H), jnp.float32),
        mesh=mesh,
        scratch_types=scratch,
        compiler_params=pltpu.CompilerParams(use_tc_tiling_on_sc=False),
    )
    def k(x_hbm, emb_hbm, pos_hbm, out_hbm, pos_v, idx_v, rows_v, *sems):
        sem_i = sems[0:_NBUF]
        sem_g = sems[_NBUF:2 * _NBUF]
        sem_s = sems[2 * _NBUF:3 * _NBUF]
        wid = lax.axis_index("s") * _NC + lax.axis_index("c")
        pltpu.sync_copy(pos_hbm, pos_v)
        b0 = wid * n_chunks

        def fire_idx(i, s):
            pltpu.async_copy(x_hbm.at[pl.ds((b0 + i) * L, L)], idx_v.at[s],
                             sem_i[s])

        def wait_idx(s):
            pltpu.make_async_copy(x_hbm.at[pl.ds(0, L)], idx_v.at[s],
                                  sem_i[s]).wait()

        def fire_gather(s):
            for off in range(0, L, 128):
                n = min(128, L - off)
                pltpu.async_copy(emb_hbm.at[idx_v.at[s, pl.ds(off, n)]],
                                 rows_v.at[s, pl.ds(off, n)], sem_g[s])

        def wait_gather(s):
            pltpu.make_async_copy(out_hbm.at[0], rows_v.at[s],
                                  sem_g[s]).wait()

        def fire_store(i, s):
            pltpu.async_copy(rows_v.at[s], out_hbm.at[b0 + i], sem_s[s])

        def wait_store(s):
            pltpu.make_async_copy(rows_v.at[s], out_hbm.at[0],
                                  sem_s[s]).wait()

        def add_pos(s):
            def add_row(r, c):
                for j in range(H // _LANES):
                    v = pos_v[r, pl.ds(j * _LANES, _LANES)]
                    plsc.addupdate(rows_v.at[s, r, pl.ds(j * _LANES, _LANES)],
                                   v)
                return c

            lax.fori_loop(0, L, add_row, 0, unroll=4)

        for j in range(_NBUF - 1):
            fire_idx(j, j)
            wait_idx(j)
            fire_gather(j)

        def step(i, s, prefetch, first=False):
            sp = (s + _NBUF - 1) % _NBUF
            if prefetch:
                fire_idx(i + (_NBUF - 1), sp)
            wait_gather(s)
            add_pos(s)
            fire_store(i, s)
            if prefetch:
                if not first:
                    wait_store(sp)
                wait_idx(sp)
                fire_gather(sp)

        step(0, 0, prefetch=True, first=True)

        n_main = n_chunks - 1 - (_NBUF - 1)

        def main_body(t, c):
            for u in range(_NBUF):
                i = 1 + t * _NBUF + u
                step(i, (1 + u) % _NBUF, prefetch=True)
            return c

        lax.fori_loop(0, n_main // _NBUF, main_body, 0)

        for i in range(n_chunks - (_NBUF - 1), n_chunks):
            step(i, i % _NBUF, prefetch=False)

        for s in range(_NBUF):
            wait_store(s)

    return k
